# baseline (speedup 1.0000x reference)
import functools

import jax
import jax.numpy as jnp
from jax import lax
from jax.experimental import pallas as pl
from jax.experimental.pallas import tpu as pltpu
from jax.experimental.pallas import tpu_sc as plsc

_NUM_CORES = 2
_NUM_SUBCORES = 16
_NUM_WORKERS = _NUM_CORES * _NUM_SUBCORES
_LANES = 16


@functools.lru_cache(maxsize=None)
def _build_sc_gather(b, n, embed_dim, max_seq_len):
    assert b % 128 == 0 and embed_dim % 8 == 0
    n_er = embed_dim // 8
    n_bc = b // 128
    slab = n_er * n_bc * 8 * 128
    half_er = n_bc // 2 * 8 * 128
    n_full = n // _NUM_WORKERS
    rem = n % _NUM_WORKERS
    n_erp = n_er // 2
    if rem:
        assert _NUM_WORKERS % rem == 0 and n_erp % (_NUM_WORKERS // rem) == 0
    mesh = plsc.VectorSubcoreMesh(
        core_axis_name="c", subcore_axis_name="s",
        num_cores=_NUM_CORES, num_subcores=_NUM_SUBCORES)

    @functools.partial(
        pl.kernel,
        out_type=jax.ShapeDtypeStruct((n * slab,), jnp.float32),
        mesh=mesh,
        scratch_types=[
            pltpu.VMEM((embed_dim * max_seq_len,), jnp.float32),
            pltpu.VMEM((b,), jnp.float32),
            pltpu.VMEM((b,), jnp.int32),
            pltpu.VMEM((2, 2 * half_er), jnp.float32),
            pltpu.SemaphoreType.DMA,
        ],
        compiler_params=pltpu.CompilerParams(needs_layout_passes=False),
    )
    def sc_kernel(pos_hbm, tab_hbm, out_hbm, tab_v, pos_v, idx_v, obuf,
                  osem):
        wid = lax.axis_index("s") * _NUM_CORES + lax.axis_index("c")

        pltpu.sync_copy(tab_hbm, tab_v)

        scale = jnp.float32(max_seq_len - 1)
        hi = jnp.int32(max_seq_len - 1)

        def out_chunks(slab_n, er0, s2):
            return [
                pltpu.make_async_copy(
                    obuf.at[s2].at[pl.ds(erl * half_er, half_er)],
                    out_hbm.at[pl.ds(slab_n * slab
                                     + (er0 + erl) * (2 * half_er)
                                     + s2 * half_er, half_er)], osem)
                for erl in range(2)]

        def load_slab(sn):
            pltpu.sync_copy(pos_hbm.at[sn], pos_v)

            @plsc.parallel_loop(0, b // _LANES, step=1, unroll=4,
                                carry=jnp.int32(0))
            def _(j, c):
                p = pos_v[pl.ds(j * _LANES, _LANES)]
                iv = (p * scale).astype(jnp.int32)
                idx_v[pl.ds(j * _LANES, _LANES)] = (
                    jnp.minimum(jnp.maximum(iv, 0), hi))
                return c

        def do_chunk(er0, s2):
            rows = [tab_v.at[pl.ds((er0 * 8 + g) * max_seq_len,
                                   max_seq_len)]
                    for g in range(16)]

            @plsc.parallel_loop(0, n_bc // 2, step=1, unroll=2,
                                carry=jnp.int32(0))
            def _(bc_loc, cc):
                base = (s2 * (n_bc // 2) + bc_loc) * 128
                ivs = [idx_v[pl.ds(base + jl * _LANES, _LANES)]
                       for jl in range(8)]

                def st(g, jl, v):
                    obuf[s2, pl.ds((g // 8) * half_er + bc_loc * 1024
                                   + (g % 8) * 128 + jl * _LANES,
                                   _LANES)] = v

                prev = [plsc.load_gather(rows[0], [ivs[jl]])
                        for jl in range(8)]
                for g in range(1, 16):
                    cur = []
                    for jl in range(8):
                        cur.append(plsc.load_gather(rows[g], [ivs[jl]]))
                        st(g - 1, jl, prev[jl])
                    prev = cur
                for jl in range(8):
                    st(15, jl, prev[jl])
                return cc

        def do_step(sn, er0, s2, h):
            @pl.when(h >= 2)
            def _():
                for c in out_chunks(sn, er0, s2):
                    c.wait()

            do_chunk(er0, s2)
            for c in out_chunks(sn, er0, s2):
                c.start()

        def do_slab(slab_i, carry):
            sn = slab_i * _NUM_WORKERS + wid
            load_slab(sn)

            def do_erp(erp, c):
                for s2 in range(2):
                    do_step(sn, erp * 2, s2,
                            slab_i * (2 * n_erp) + erp * 2 + s2)
                return c

            lax.fori_loop(0, n_erp, do_erp, 0)
            return carry

        lax.fori_loop(0, n_full, do_slab, 0)

        if rem:
            tps = _NUM_WORKERS // rem
            erp_per = n_erp // tps
            sn = n_full * _NUM_WORKERS + wid // tps
            erp_base = (wid % tps) * erp_per
            load_slab(sn)
            for erp_loc in range(erp_per):
                er0 = (erp_base + erp_loc) * 2
                for s2 in range(2):
                    do_step(sn, er0, s2, jnp.int32(2))

        for _ in range(4):
            pltpu.make_async_copy(
                obuf.at[0].at[pl.ds(0, half_er)],
                out_hbm.at[pl.ds(0, half_er)], osem).wait()

    return sc_kernel


def kernel(positions, pos_embeddings):
    max_seq_len, embed_dim = pos_embeddings.shape
    b, n, _ = positions.shape
    pos_t = positions.reshape(b, n).T
    tab_t = pos_embeddings.T.reshape(-1)
    sc = _build_sc_gather(b, n, embed_dim, max_seq_len)
    flat = sc(pos_t, tab_t)
    o5 = flat.reshape(n, embed_dim // 8, b // 128, 8, 128)
    return o5.transpose(2, 4, 0, 1, 3).reshape(b, n, embed_dim)


# speedup vs baseline: 1.0787x; 1.0787x over previous
import functools

import jax
import jax.numpy as jnp
from jax import lax
from jax.experimental import pallas as pl
from jax.experimental.pallas import tpu as pltpu
from jax.experimental.pallas import tpu_sc as plsc

_NUM_CORES = 2
_NUM_SUBCORES = 16
_NUM_WORKERS = _NUM_CORES * _NUM_SUBCORES
_LANES = 16


@functools.lru_cache(maxsize=None)
def _build_sc_gather(b, n, embed_dim, max_seq_len):
    assert b % 128 == 0 and embed_dim % 8 == 0
    n_er = embed_dim // 8
    n_bc = b // 128
    slab = n_er * n_bc * 8 * 128
    half_er = n_bc // 2 * 8 * 128
    n_full = n // _NUM_WORKERS
    rem = n % _NUM_WORKERS
    if rem:
        assert _NUM_WORKERS % rem == 0 and n_er % (_NUM_WORKERS // rem) == 0
    mesh = plsc.VectorSubcoreMesh(
        core_axis_name="c", subcore_axis_name="s",
        num_cores=_NUM_CORES, num_subcores=_NUM_SUBCORES)

    @functools.partial(
        pl.kernel,
        out_type=jax.ShapeDtypeStruct((n * slab,), jnp.float32),
        mesh=mesh,
        scratch_types=[
            pltpu.VMEM((embed_dim * max_seq_len,), jnp.float32),
            pltpu.VMEM((b,), jnp.float32),
            pltpu.VMEM((b,), jnp.int32),
            pltpu.VMEM((2, half_er), jnp.float32),
            pltpu.SemaphoreType.DMA,
        ],
        compiler_params=pltpu.CompilerParams(needs_layout_passes=False),
    )
    def sc_kernel(pos_hbm, tab_hbm, out_hbm, tab_v, pos_v, idx_v, obuf,
                  osem):
        wid = lax.axis_index("s") * _NUM_CORES + lax.axis_index("c")

        pltpu.sync_copy(tab_hbm, tab_v)

        scale = jnp.float32(max_seq_len - 1)
        hi = jnp.int32(max_seq_len - 1)

        def out_chunk(slab_n, er, s2):
            return pltpu.make_async_copy(
                obuf.at[s2],
                out_hbm.at[pl.ds(slab_n * slab + er * (2 * half_er)
                                 + s2 * half_er, half_er)], osem)

        def load_slab(sn):
            pltpu.sync_copy(pos_hbm.at[sn], pos_v)

            @plsc.parallel_loop(0, b // _LANES, step=1, unroll=4,
                                carry=jnp.int32(0))
            def _(j, c):
                p = pos_v[pl.ds(j * _LANES, _LANES)]
                iv = (p * scale).astype(jnp.int32)
                idx_v[pl.ds(j * _LANES, _LANES)] = (
                    jnp.minimum(jnp.maximum(iv, 0), hi))
                return c

        def do_chunk(er, s2):
            rows = [tab_v.at[pl.ds((er * 8 + es) * max_seq_len,
                                   max_seq_len)]
                    for es in range(8)]

            @plsc.parallel_loop(0, n_bc // 2, step=1, unroll=2,
                                carry=jnp.int32(0))
            def _(bc_loc, cc):
                base = (s2 * (n_bc // 2) + bc_loc) * 128
                ivs = [idx_v[pl.ds(base + jl * _LANES, _LANES)]
                       for jl in range(8)]

                def st(es, jl, v):
                    obuf[s2, pl.ds(bc_loc * 1024 + es * 128
                                   + jl * _LANES, _LANES)] = v

                prev = [plsc.load_gather(rows[0], [ivs[jl]])
                        for jl in range(8)]
                for es in range(1, 8):
                    cur = []
                    for jl in range(8):
                        cur.append(plsc.load_gather(rows[es], [ivs[jl]]))
                        st(es - 1, jl, prev[jl])
                    prev = cur
                for jl in range(8):
                    st(7, jl, prev[jl])
                return cc

        def do_slab(slab_i, carry):
            sn = slab_i * _NUM_WORKERS + wid
            load_slab(sn)

            def do_er(er, c):
                for s2 in range(2):
                    h = slab_i * (2 * n_er) + er * 2 + s2

                    @pl.when(h >= 2)
                    def _():
                        out_chunk(sn, er, s2).wait()

                    do_chunk(er, s2)
                    out_chunk(sn, er, s2).start()
                return c

            lax.fori_loop(0, n_er, do_er, 0)
            return carry

        lax.fori_loop(0, n_full, do_slab, 0)

        if rem:
            tps = _NUM_WORKERS // rem
            er_per = n_er // tps
            sn = n_full * _NUM_WORKERS + wid // tps
            er_base = (wid % tps) * er_per
            load_slab(sn)
            for er_loc in range(er_per):
                er = er_base + er_loc
                for s2 in range(2):
                    out_chunk(sn, er, s2).wait()
                    do_chunk(er, s2)
                    out_chunk(sn, er, s2).start()

        for _ in range(2):
            pltpu.make_async_copy(
                obuf.at[0], out_hbm.at[pl.ds(0, half_er)], osem).wait()

    return sc_kernel


def kernel(positions, pos_embeddings):
    max_seq_len, embed_dim = pos_embeddings.shape
    b, n, _ = positions.shape
    pos_t = positions.reshape(b, n).T
    tab_t = pos_embeddings.T.reshape(-1)
    sc = _build_sc_gather(b, n, embed_dim, max_seq_len)
    flat = sc(pos_t, tab_t)
    o5 = flat.reshape(n, embed_dim // 8, b // 128, 8, 128)
    return o5.transpose(2, 4, 0, 1, 3).reshape(b, n, embed_dim)


# speedup vs baseline: 1.0825x; 1.0035x over previous
import functools

import jax
import jax.numpy as jnp
from jax import lax
from jax.experimental import pallas as pl
from jax.experimental.pallas import tpu as pltpu
from jax.experimental.pallas import tpu_sc as plsc

_NUM_CORES = 2
_NUM_SUBCORES = 16
_NUM_WORKERS = _NUM_CORES * _NUM_SUBCORES
_LANES = 16


@functools.lru_cache(maxsize=None)
def _build_sc_gather(b, n, embed_dim, max_seq_len):
    assert b % 128 == 0 and embed_dim % 8 == 0
    n_er = embed_dim // 8
    n_bc = b // 128
    slab = n_er * n_bc * 8 * 128
    half_er = n_bc // 2 * 8 * 128
    n_full = n // _NUM_WORKERS
    rem = n % _NUM_WORKERS
    if rem:
        assert _NUM_WORKERS % rem == 0 and n_er % (_NUM_WORKERS // rem) == 0
    mesh = plsc.VectorSubcoreMesh(
        core_axis_name="c", subcore_axis_name="s",
        num_cores=_NUM_CORES, num_subcores=_NUM_SUBCORES)

    @functools.partial(
        pl.kernel,
        out_type=jax.ShapeDtypeStruct((n * slab,), jnp.float32),
        mesh=mesh,
        scratch_types=[
            pltpu.VMEM((embed_dim * max_seq_len,), jnp.float32),
            pltpu.VMEM((b,), jnp.float32),
            pltpu.VMEM((b,), jnp.int32),
            pltpu.VMEM((2, half_er), jnp.float32),
            pltpu.SemaphoreType.DMA,
        ],
        compiler_params=pltpu.CompilerParams(needs_layout_passes=False),
    )
    def sc_kernel(pos_hbm, tab_hbm, out_hbm, tab_v, pos_v, idx_v, obuf,
                  osem):
        wid = lax.axis_index("s") * _NUM_CORES + lax.axis_index("c")

        pltpu.sync_copy(tab_hbm, tab_v)

        scale = jnp.float32(max_seq_len - 1)
        hi = jnp.int32(max_seq_len - 1)

        def out_chunk(slab_n, er, s2):
            return pltpu.make_async_copy(
                obuf.at[s2],
                out_hbm.at[pl.ds(slab_n * slab + er * (2 * half_er)
                                 + s2 * half_er, half_er)], osem)

        def load_slab(sn):
            pltpu.sync_copy(pos_hbm.at[sn], pos_v)

            @plsc.parallel_loop(0, b // _LANES, step=1, unroll=4,
                                carry=jnp.int32(0))
            def _(j, c):
                p = pos_v[pl.ds(j * _LANES, _LANES)]
                iv = (p * scale).astype(jnp.int32)
                idx_v[pl.ds(j * _LANES, _LANES)] = (
                    jnp.minimum(jnp.maximum(iv, 0), hi))
                return c

        def do_chunk(er, s2):
            rows = [tab_v.at[pl.ds((er * 8 + es) * max_seq_len,
                                   max_seq_len)]
                    for es in range(8)]

            @plsc.parallel_loop(0, n_bc // 2, step=1, unroll=1,
                                carry=jnp.int32(0))
            def _(bc_loc, cc):
                base = (s2 * (n_bc // 2) + bc_loc) * 128
                ivs = [idx_v[pl.ds(base + jl * _LANES, _LANES)]
                       for jl in range(8)]

                def st(es, jl, v):
                    obuf[s2, pl.ds(bc_loc * 1024 + es * 128
                                   + jl * _LANES, _LANES)] = v

                prev = [plsc.load_gather(rows[0], [ivs[jl]])
                        for jl in range(8)]
                for es in range(1, 8):
                    cur = []
                    for jl in range(8):
                        cur.append(plsc.load_gather(rows[es], [ivs[jl]]))
                        st(es - 1, jl, prev[jl])
                    prev = cur
                for jl in range(8):
                    st(7, jl, prev[jl])
                return cc

        def do_slab(slab_i, carry):
            sn = slab_i * _NUM_WORKERS + wid
            load_slab(sn)

            def do_er(er, c):
                for s2 in range(2):
                    h = slab_i * (2 * n_er) + er * 2 + s2

                    @pl.when(h >= 2)
                    def _():
                        out_chunk(sn, er, s2).wait()

                    do_chunk(er, s2)
                    out_chunk(sn, er, s2).start()
                return c

            lax.fori_loop(0, n_er, do_er, 0)
            return carry

        lax.fori_loop(0, n_full, do_slab, 0)

        if rem:
            tps = _NUM_WORKERS // rem
            er_per = n_er // tps
            sn = n_full * _NUM_WORKERS + wid // tps
            er_base = (wid % tps) * er_per
            load_slab(sn)
            for er_loc in range(er_per):
                er = er_base + er_loc
                for s2 in range(2):
                    out_chunk(sn, er, s2).wait()
                    do_chunk(er, s2)
                    out_chunk(sn, er, s2).start()

        for _ in range(2):
            pltpu.make_async_copy(
                obuf.at[0], out_hbm.at[pl.ds(0, half_er)], osem).wait()

    return sc_kernel


def kernel(positions, pos_embeddings):
    max_seq_len, embed_dim = pos_embeddings.shape
    b, n, _ = positions.shape
    pos_t = positions.reshape(b, n).T
    tab_t = pos_embeddings.T.reshape(-1)
    sc = _build_sc_gather(b, n, embed_dim, max_seq_len)
    flat = sc(pos_t, tab_t)
    o5 = flat.reshape(n, embed_dim // 8, b // 128, 8, 128)
    return o5.transpose(2, 4, 0, 1, 3).reshape(b, n, embed_dim)
